# baseline (speedup 1.0000x reference)
import jax
import jax.numpy as jnp
from jax import lax
from jax.experimental import pallas as pl
from jax.experimental.pallas import tpu as pltpu

VOCAB_PAD = 1024
BLK = 512


def _body(x_ref, tbl_ref, out_ref, W_ref, tot_ref, sc_ref):
    i = pl.program_id(0)

    @pl.when(i == 0)
    def _init():
        W_ref[...] = jnp.zeros((VOCAB_PAD, 256), jnp.bfloat16)
        tbl = tbl_ref[...]
        shifted = jnp.concatenate([tbl[:, :1], tbl[:, :127]], axis=1)
        ne = (tbl != shifted).astype(jnp.float32)
        r = lax.broadcasted_iota(jnp.int32, (128, 128), 0)
        c = lax.broadcasted_iota(jnp.int32, (128, 128), 1)
        M = (r <= c).astype(jnp.float32)
        P = jnp.dot(ne, M, preferred_element_type=jnp.float32)
        W_ref[0:1000, 0:128] = P.astype(jnp.bfloat16)
        fbits = lax.bitcast_convert_type(tbl[:, 0:1], jnp.int32)
        lbits = lax.bitcast_convert_type(tbl[:, 127:128], jnp.int32)
        for k in range(4):
            fb = ((fbits >> (8 * k)) & 255).astype(jnp.bfloat16)
            lb = ((lbits >> (8 * k)) & 255).astype(jnp.bfloat16)
            W_ref[0:1000, 128 + k:129 + k] = fb
            W_ref[0:1000, 132 + k:133 + k] = lb
        tot_ref[0] = 0
        for k in range(5):
            sc_ref[k] = 0.0

    xv = x_ref[...]
    iota_v = lax.broadcasted_iota(jnp.int16, (BLK, VOCAB_PAD), 1)
    oh = jnp.where(xv.astype(jnp.int16) == iota_v,
                   jnp.bfloat16(1), jnp.bfloat16(0))
    G = jnp.dot(oh, W_ref[...], preferred_element_type=jnp.float32)
    Gp = G[:, 0:128]
    c_col = Gp[:, 127:128]
    fb = G[:, 128:132]
    lb = G[:, 132:136]

    prev_c = sc_ref[0]
    prev_lb = jnp.concatenate(
        [jnp.full((1, 1), sc_ref[1 + k], jnp.float32) for k in range(4)], axis=1)
    c_sh = jnp.concatenate([jnp.full((1, 1), prev_c, jnp.float32), c_col[:-1, :]],
                           axis=0)
    lb_sh = jnp.concatenate([prev_lb, lb[:-1, :]], axis=0)
    neq = jnp.sum(jnp.abs(fb - lb_sh), axis=1, keepdims=True)
    val = c_sh + (neq > 0.5).astype(jnp.float32)
    row = lax.broadcasted_iota(jnp.int32, (BLK, 1), 0)
    val = jnp.where((i == 0) & (row == 0), 0.0, val)

    rT = lax.broadcasted_iota(jnp.int16, (BLK, BLK), 0)
    cT = lax.broadcasted_iota(jnp.int16, (BLK, BLK), 1)
    L = jnp.where(cT <= rT, jnp.bfloat16(1), jnp.bfloat16(0))
    S_rel = jnp.dot(L, val.astype(jnp.bfloat16),
                    preferred_element_type=jnp.float32)
    S = tot_ref[0] + S_rel.astype(jnp.int32)

    out_ref[...] = Gp.astype(jnp.int32) + S

    tot_ref[0] = tot_ref[0] + jnp.sum(val).astype(jnp.int32)
    sc_ref[0] = jnp.sum(c_col[BLK - 1:BLK, :])
    for k in range(4):
        sc_ref[1 + k] = jnp.sum(lb[BLK - 1:BLK, k:k + 1])


def kernel(x, table):
    B, Lx = x.shape
    T = B * Lx
    x2 = x.reshape(T, 1)
    grid = T // BLK
    out = pl.pallas_call(
        _body,
        grid=(grid,),
        in_specs=[
            pl.BlockSpec((BLK, 1), lambda i: (i, 0)),
            pl.BlockSpec((1000, 128), lambda i: (0, 0)),
        ],
        out_specs=pl.BlockSpec((BLK, 128), lambda i: (i, 0)),
        out_shape=jax.ShapeDtypeStruct((T, 128), jnp.int32),
        scratch_shapes=[
            pltpu.VMEM((VOCAB_PAD, 256), jnp.bfloat16),
            pltpu.SMEM((1,), jnp.int32),
            pltpu.SMEM((8,), jnp.float32),
        ],
    )(x2, table)
    return out.reshape(B, Lx, 128)


# speedup vs baseline: 1.2052x; 1.2052x over previous
import jax
import jax.numpy as jnp
from jax import lax
from jax.experimental import pallas as pl
from jax.experimental.pallas import tpu as pltpu
from jax.experimental.pallas import tpu_sc as plsc

T_TOK = 204800
NW = 32
TPW = T_TOK // NW
NGRP = TPW // 128
KBLK = 2048


def _prep_body(tbl_ref, p_ref, meta_ref):
    tbl = tbl_ref[...]
    shifted = jnp.concatenate([tbl[:, :1], tbl[:, :127]], axis=1)
    ne = (tbl != shifted).astype(jnp.float32)
    r = lax.broadcasted_iota(jnp.int32, (128, 128), 0)
    c = lax.broadcasted_iota(jnp.int32, (128, 128), 1)
    M = (r <= c).astype(jnp.float32)
    P = jnp.dot(ne, M, preferred_element_type=jnp.float32)
    p_ref[...] = P.astype(jnp.int32)
    meta_ref[...] = jnp.zeros((1000, 8), jnp.int32)
    meta_ref[:, 0:1] = P[:, 127:128].astype(jnp.int32)
    meta_ref[:, 1:2] = lax.bitcast_convert_type(tbl[:, 0:1], jnp.int32)
    meta_ref[:, 2:3] = lax.bitcast_convert_type(tbl[:, 127:128], jnp.int32)


def _prep(table):
    return pl.pallas_call(
        _prep_body,
        out_shape=(
            jax.ShapeDtypeStruct((1000, 128), jnp.int32),
            jax.ShapeDtypeStruct((1000, 8), jnp.int32),
        ),
    )(table)


def _wid():
    return lax.axis_index("s") * 2 + lax.axis_index("c")


def _k1_body(x2_hbm, xp2_hbm, c_hbm, f_hbm, l_hbm, val_hbm,
             idx_v, idxp_v, cs_v, fs_v, ls_v, vv_v, sem):
    wid = _wid()
    base = wid * TPW
    grow0 = wid * NGRP
    pltpu.sync_copy(x2_hbm.at[wid], idx_v)
    pltpu.sync_copy(xp2_hbm.at[wid], idxp_v)

    for g in range(NGRP):
        pltpu.async_copy(c_hbm.at[idxp_v.at[g]], cs_v.at[pl.ds(g * 128, 128)],
                         sem)
        pltpu.async_copy(l_hbm.at[idxp_v.at[g]], ls_v.at[pl.ds(g * 128, 128)],
                         sem)
        pltpu.async_copy(f_hbm.at[idx_v.at[g]], fs_v.at[pl.ds(g * 128, 128)],
                         sem)
    for g in range(NGRP):
        pltpu.make_async_copy(c_hbm.at[idxp_v.at[g]],
                              cs_v.at[pl.ds(g * 128, 128)], sem).wait()
        pltpu.make_async_copy(l_hbm.at[idxp_v.at[g]],
                              ls_v.at[pl.ds(g * 128, 128)], sem).wait()
        pltpu.make_async_copy(f_hbm.at[idx_v.at[g]],
                              fs_v.at[pl.ds(g * 128, 128)], sem).wait()

    def step(j, carry):
        sl = pl.ds(j * 16, 16)
        val = cs_v[sl] + jnp.where(ls_v[sl] != fs_v[sl], 1, 0)
        vv_v[sl] = val
        return carry

    lax.fori_loop(0, TPW // 16, step, jnp.int32(0))
    pltpu.sync_copy(vv_v, val_hbm.at[pl.ds(base, TPW)])


def _k2_body(v_ref, s_ref, tot_ref):
    i = pl.program_id(0)

    @pl.when(i == 0)
    def _():
        tot_ref[0] = 0

    val = v_ref[...].astype(jnp.float32)
    row = lax.broadcasted_iota(jnp.int32, (KBLK, 1), 0)
    val = jnp.where((i == 0) & (row == 0), 0.0, val)
    rT = lax.broadcasted_iota(jnp.int16, (KBLK, KBLK), 0)
    cT = lax.broadcasted_iota(jnp.int16, (KBLK, KBLK), 1)
    L = jnp.where(cT <= rT, jnp.bfloat16(1), jnp.bfloat16(0))
    s_rel = jnp.dot(L, val.astype(jnp.bfloat16),
                    preferred_element_type=jnp.float32)
    s_ref[...] = tot_ref[0] + s_rel.astype(jnp.int32)
    tot_ref[0] = tot_ref[0] + jnp.sum(val).astype(jnp.int32)


def _cumsum(val):
    return pl.pallas_call(
        _k2_body,
        grid=(T_TOK // KBLK,),
        in_specs=[pl.BlockSpec((KBLK, 1), lambda i: (i, 0))],
        out_specs=pl.BlockSpec((KBLK, 1), lambda i: (i, 0)),
        out_shape=jax.ShapeDtypeStruct((T_TOK, 1), jnp.int32),
        scratch_shapes=[pltpu.SMEM((1,), jnp.int32)],
    )(val.reshape(T_TOK, 1))


def _k3_body(x2_hbm, p_hbm, s_hbm, out_hbm,
             idx_v, sv_v, rows0, rows1, gs0, gs1, ss0, ss1):
    wid = _wid()
    base = wid * TPW
    pltpu.sync_copy(x2_hbm.at[wid], idx_v)
    pltpu.sync_copy(s_hbm.at[pl.ds(base, TPW)], sv_v)

    rows = (rows0, rows1)
    gsem = (gs0, gs1)
    ssem = (ss0, ss1)

    def wait_gather(b):
        pltpu.make_async_copy(p_hbm.at[pl.ds(0, 128)], rows[b],
                              gsem[b]).wait()

    def wait_scatter(b, g):
        pltpu.make_async_copy(rows[b],
                              out_hbm.at[pl.ds(base + (g * 128), 128)],
                              ssem[b]).wait()

    def process(g, b):
        def tok16(j, carry):
            s_vec = sv_v[pl.ds(g * 128 + j * 16, 16)]
            for k in range(16):
                spl = jnp.full((16,), s_vec[k], jnp.int32)
                t = j * 16 + k
                for gg in range(8):
                    sl = (t, pl.ds(gg * 16, 16))
                    rows[b][sl] = rows[b][sl] + spl
            return carry

        lax.fori_loop(0, 8, tok16, jnp.int32(0))

    def body(k, carry):
        for b in range(2):
            g = k * 2 + b
            wait_gather(b)
            process(g, b)
            pltpu.async_copy(rows[b], out_hbm.at[pl.ds(base + (g * 128), 128)],
                             ssem[b])
            wait_scatter(b, g)
            pltpu.async_copy(p_hbm.at[idx_v.at[g + 2]], rows[b], gsem[b])
        return carry

    for b in range(2):
        pltpu.async_copy(p_hbm.at[idx_v.at[b]], rows[b], gsem[b])
    lax.fori_loop(0, (NGRP - 2) // 2, body, jnp.int32(0))
    for b in range(2):
        g = NGRP - 2 + b
        wait_gather(b)
        process(g, b)
        pltpu.async_copy(rows[b], out_hbm.at[pl.ds(base + (g * 128), 128)],
                         ssem[b])
        wait_scatter(b, g)


def kernel(x, table):
    B, Lx = x.shape
    p_arr, meta = _prep(table)
    c1 = meta[:, 0]
    f1 = meta[:, 1]
    l1 = meta[:, 2]
    x_flat = x.reshape(T_TOK)
    x_prev = jnp.concatenate([x_flat[:1], x_flat[:-1]])
    x2 = x_flat.reshape(NW, NGRP, 128)
    xp2 = x_prev.reshape(NW, NGRP, 128)

    mesh = plsc.VectorSubcoreMesh(core_axis_name="c", subcore_axis_name="s")

    k1 = pl.kernel(
        _k1_body,
        mesh=mesh,
        out_type=jax.ShapeDtypeStruct((T_TOK,), jnp.int32),
        scratch_types=[
            pltpu.VMEM((NGRP, 128), jnp.int32),
            pltpu.VMEM((NGRP, 128), jnp.int32),
            pltpu.VMEM((TPW,), jnp.int32),
            pltpu.VMEM((TPW,), jnp.int32),
            pltpu.VMEM((TPW,), jnp.int32),
            pltpu.VMEM((TPW,), jnp.int32),
            pltpu.SemaphoreType.DMA,
        ],
    )
    val = k1(x2, xp2, c1, f1, l1)

    s_tok = _cumsum(val).reshape(T_TOK)

    k3 = pl.kernel(
        _k3_body,
        mesh=mesh,
        out_type=jax.ShapeDtypeStruct((T_TOK, 128), jnp.int32),
        scratch_types=[
            pltpu.VMEM((NGRP, 128), jnp.int32),
            pltpu.VMEM((TPW,), jnp.int32),
            pltpu.VMEM((128, 128), jnp.int32),
            pltpu.VMEM((128, 128), jnp.int32),
            pltpu.SemaphoreType.DMA,
            pltpu.SemaphoreType.DMA,
            pltpu.SemaphoreType.DMA,
            pltpu.SemaphoreType.DMA,
        ],
    )
    out = k3(x2, p_arr, s_tok)
    return out.reshape(B, Lx, 128)


# speedup vs baseline: 1.5101x; 1.2530x over previous
import jax
import jax.numpy as jnp
from jax import lax
from jax.experimental import pallas as pl
from jax.experimental.pallas import tpu as pltpu
from jax.experimental.pallas import tpu_sc as plsc

T_TOK = 204800
NW = 32
TPW = T_TOK // NW
NGRP = TPW // 128
KBLK = 2048


def _prep_body(tbl_ref, p_ref, meta_ref):
    tbl = tbl_ref[...]
    shifted = jnp.concatenate([tbl[:, :1], tbl[:, :127]], axis=1)
    ne = (tbl != shifted).astype(jnp.float32)
    r = lax.broadcasted_iota(jnp.int32, (128, 128), 0)
    c = lax.broadcasted_iota(jnp.int32, (128, 128), 1)
    M = (r <= c).astype(jnp.float32)
    P = jnp.dot(ne, M, preferred_element_type=jnp.float32)
    p_ref[...] = P.astype(jnp.int32)
    meta_ref[...] = jnp.zeros((1000, 8), jnp.int32)
    meta_ref[:, 0:1] = P[:, 127:128].astype(jnp.int32)
    meta_ref[:, 1:2] = lax.bitcast_convert_type(tbl[:, 0:1], jnp.int32)
    meta_ref[:, 2:3] = lax.bitcast_convert_type(tbl[:, 127:128], jnp.int32)


def _prep(table):
    return pl.pallas_call(
        _prep_body,
        out_shape=(
            jax.ShapeDtypeStruct((1000, 128), jnp.int32),
            jax.ShapeDtypeStruct((1000, 8), jnp.int32),
        ),
    )(table)


def _vt_body(meta_ref, frow_ref, vt_ref):
    c_col = meta_ref[:, 0:1]
    l_col = meta_ref[:, 2:3]
    f_row = frow_ref[...]
    vt_ref[...] = c_col + jnp.where(l_col != f_row, 1, 0)


def _valtab(meta, f_row):
    return pl.pallas_call(
        _vt_body,
        out_shape=jax.ShapeDtypeStruct((1000, 1000), jnp.int32),
    )(meta, f_row)


def _wid():
    return lax.axis_index("s") * 2 + lax.axis_index("c")


def _k1_body(x2_hbm, xp2_hbm, vt_hbm, val_hbm,
             idx_v, idxp_v, pidx_v, vv_v, sem):
    wid = _wid()
    base = wid * TPW
    pltpu.sync_copy(x2_hbm.at[wid], idx_v)
    pltpu.sync_copy(xp2_hbm.at[wid], idxp_v)

    def mkidx(g, carry):
        def sub(j, carry2):
            sl = (g, pl.ds(j * 16, 16))
            pidx_v[sl] = idxp_v[sl] * 1000 + idx_v[sl]
            return carry2
        return lax.fori_loop(0, 8, sub, carry)

    lax.fori_loop(0, NGRP, mkidx, jnp.int32(0))

    for g in range(NGRP):
        pltpu.async_copy(vt_hbm.at[pidx_v.at[g]], vv_v.at[pl.ds(g * 128, 128)],
                         sem)
    for g in range(NGRP):
        pltpu.make_async_copy(vt_hbm.at[pidx_v.at[g]],
                              vv_v.at[pl.ds(g * 128, 128)], sem).wait()
    pltpu.sync_copy(vv_v, val_hbm.at[pl.ds(base, TPW)])


def _k2_body(v_ref, s_ref, lt_ref, tot_ref):
    i = pl.program_id(0)

    @pl.when(i == 0)
    def _():
        tot_ref[0] = 0
        rT = lax.broadcasted_iota(jnp.int16, (KBLK, KBLK), 0)
        cT = lax.broadcasted_iota(jnp.int16, (KBLK, KBLK), 1)
        lt_ref[...] = jnp.where(cT <= rT, jnp.bfloat16(1), jnp.bfloat16(0))

    val = v_ref[...].astype(jnp.float32)
    row = lax.broadcasted_iota(jnp.int32, (KBLK, 1), 0)
    val = jnp.where((i == 0) & (row == 0), 0.0, val)
    s_rel = jnp.dot(lt_ref[...], val.astype(jnp.bfloat16),
                    preferred_element_type=jnp.float32)
    s_ref[...] = tot_ref[0] + s_rel.astype(jnp.int32)
    tot_ref[0] = tot_ref[0] + jnp.sum(val).astype(jnp.int32)


def _cumsum(val):
    return pl.pallas_call(
        _k2_body,
        grid=(T_TOK // KBLK,),
        in_specs=[pl.BlockSpec((KBLK, 1), lambda i: (i, 0))],
        out_specs=pl.BlockSpec((KBLK, 1), lambda i: (i, 0)),
        out_shape=jax.ShapeDtypeStruct((T_TOK, 1), jnp.int32),
        scratch_shapes=[pltpu.VMEM((KBLK, KBLK), jnp.bfloat16),
                        pltpu.SMEM((1,), jnp.int32)],
    )(val.reshape(T_TOK, 1))


def _k3_body(x2_hbm, p_hbm, s_hbm, out_hbm,
             idx_v, sv_v, rows0, rows1, gs0, gs1, ss0, ss1):
    wid = _wid()
    base = wid * TPW
    pltpu.sync_copy(x2_hbm.at[wid], idx_v)
    pltpu.sync_copy(s_hbm.at[pl.ds(base, TPW)], sv_v)

    rows = (rows0, rows1)
    gsem = (gs0, gs1)
    ssem = (ss0, ss1)

    def wait_gather(b):
        pltpu.make_async_copy(p_hbm.at[pl.ds(0, 128)], rows[b],
                              gsem[b]).wait()

    def wait_scatter(b, g):
        pltpu.make_async_copy(rows[b],
                              out_hbm.at[pl.ds(base + (g * 128), 128)],
                              ssem[b]).wait()

    def process(g, b):
        def tok16(j, carry):
            s_vec = sv_v[pl.ds(g * 128 + j * 16, 16)]
            for k in range(16):
                spl = jnp.full((16,), s_vec[k], jnp.int32)
                t = j * 16 + k
                for gg in range(8):
                    sl = (t, pl.ds(gg * 16, 16))
                    rows[b][sl] = rows[b][sl] + spl
            return carry

        lax.fori_loop(0, 8, tok16, jnp.int32(0))

    def body(k, carry):
        for b in range(2):
            g = k * 2 + b
            wait_gather(b)
            process(g, b)
            pltpu.async_copy(rows[b], out_hbm.at[pl.ds(base + (g * 128), 128)],
                             ssem[b])
            wait_scatter(b, g)
            pltpu.async_copy(p_hbm.at[idx_v.at[g + 2]], rows[b], gsem[b])
        return carry

    for b in range(2):
        pltpu.async_copy(p_hbm.at[idx_v.at[b]], rows[b], gsem[b])
    lax.fori_loop(0, (NGRP - 2) // 2, body, jnp.int32(0))
    for b in range(2):
        g = NGRP - 2 + b
        wait_gather(b)
        process(g, b)
        pltpu.async_copy(rows[b], out_hbm.at[pl.ds(base + (g * 128), 128)],
                         ssem[b])
        wait_scatter(b, g)


def kernel(x, table):
    B, Lx = x.shape
    p_arr, meta = _prep(table)
    vt = _valtab(meta, meta[:, 1].reshape(1, 1000)).reshape(1000000)
    x_flat = x.reshape(T_TOK)
    x_prev = jnp.concatenate([x_flat[:1], x_flat[:-1]])
    x2 = x_flat.reshape(NW, NGRP, 128)
    xp2 = x_prev.reshape(NW, NGRP, 128)

    mesh = plsc.VectorSubcoreMesh(core_axis_name="c", subcore_axis_name="s")

    k1 = pl.kernel(
        _k1_body,
        mesh=mesh,
        out_type=jax.ShapeDtypeStruct((T_TOK,), jnp.int32),
        scratch_types=[
            pltpu.VMEM((NGRP, 128), jnp.int32),
            pltpu.VMEM((NGRP, 128), jnp.int32),
            pltpu.VMEM((NGRP, 128), jnp.int32),
            pltpu.VMEM((TPW,), jnp.int32),
            pltpu.SemaphoreType.DMA,
        ],
    )
    val = k1(x2, xp2, vt)

    s_tok = _cumsum(val).reshape(T_TOK)

    k3 = pl.kernel(
        _k3_body,
        mesh=mesh,
        out_type=jax.ShapeDtypeStruct((T_TOK, 128), jnp.int32),
        scratch_types=[
            pltpu.VMEM((NGRP, 128), jnp.int32),
            pltpu.VMEM((TPW,), jnp.int32),
            pltpu.VMEM((128, 128), jnp.int32),
            pltpu.VMEM((128, 128), jnp.int32),
            pltpu.SemaphoreType.DMA,
            pltpu.SemaphoreType.DMA,
            pltpu.SemaphoreType.DMA,
            pltpu.SemaphoreType.DMA,
        ],
    )
    out = k3(x2, p_arr, s_tok)
    return out.reshape(B, Lx, 128)


# speedup vs baseline: 3.2033x; 2.1212x over previous
import jax
import jax.numpy as jnp
from jax import lax
from jax.experimental import pallas as pl
from jax.experimental.pallas import tpu as pltpu
from jax.experimental.pallas import tpu_sc as plsc

T_TOK = 204800
NW = 32
TPW = T_TOK // NW
NGRP = TPW // 128
KBLK = 2048


def _prep_body(tbl_ref, p_ref, meta_ref):
    tbl = tbl_ref[...]
    shifted = jnp.concatenate([tbl[:, :1], tbl[:, :127]], axis=1)
    ne = (tbl != shifted).astype(jnp.float32)
    r = lax.broadcasted_iota(jnp.int32, (128, 128), 0)
    c = lax.broadcasted_iota(jnp.int32, (128, 128), 1)
    M = (r <= c).astype(jnp.float32)
    P = jnp.dot(ne, M, preferred_element_type=jnp.float32)
    p_ref[...] = P.astype(jnp.int32)
    meta_ref[...] = jnp.zeros((1000, 8), jnp.int32)
    meta_ref[:, 0:1] = P[:, 127:128].astype(jnp.int32)
    meta_ref[:, 1:2] = lax.bitcast_convert_type(tbl[:, 0:1], jnp.int32)
    meta_ref[:, 2:3] = lax.bitcast_convert_type(tbl[:, 127:128], jnp.int32)


def _prep(table):
    return pl.pallas_call(
        _prep_body,
        out_shape=(
            jax.ShapeDtypeStruct((1000, 128), jnp.int32),
            jax.ShapeDtypeStruct((1000, 8), jnp.int32),
        ),
    )(table)


def _vt_body(meta_ref, frow_ref, vt_ref):
    c_col = meta_ref[:, 0:1]
    l_col = meta_ref[:, 2:3]
    f_row = frow_ref[...]
    vt_ref[...] = c_col + jnp.where(l_col != f_row, 1, 0)


def _valtab(meta, f_row):
    return pl.pallas_call(
        _vt_body,
        out_shape=jax.ShapeDtypeStruct((1000, 1000), jnp.int32),
    )(meta, f_row)


def _wid():
    return lax.axis_index("s") * 2 + lax.axis_index("c")


def _k1_body(x2_hbm, xp2_hbm, vt_hbm, val_hbm,
             idx_v, idxp_v, pidx_v, vv_v, sem):
    wid = _wid()
    base = wid * TPW
    pltpu.sync_copy(x2_hbm.at[wid], idx_v)
    pltpu.sync_copy(xp2_hbm.at[wid], idxp_v)

    def mkidx(g, carry):
        def sub(j, carry2):
            sl = (g, pl.ds(j * 16, 16))
            pidx_v[sl] = idxp_v[sl] * 1000 + idx_v[sl]
            return carry2
        return lax.fori_loop(0, 8, sub, carry)

    lax.fori_loop(0, NGRP, mkidx, jnp.int32(0))

    for g in range(NGRP):
        pltpu.async_copy(vt_hbm.at[pidx_v.at[g]], vv_v.at[pl.ds(g * 128, 128)],
                         sem)
    for g in range(NGRP):
        pltpu.make_async_copy(vt_hbm.at[pidx_v.at[g]],
                              vv_v.at[pl.ds(g * 128, 128)], sem).wait()
    pltpu.sync_copy(vv_v, val_hbm.at[pl.ds(base, TPW)])


def _k2_body(v_ref, s_ref, tot_ref):
    i = pl.program_id(0)

    @pl.when(i == 0)
    def _():
        tot_ref[0] = 0

    v = v_ref[0]
    rI = lax.broadcasted_iota(jnp.int32, (16, 128), 0)
    cI = lax.broadcasted_iota(jnp.int32, (16, 128), 1)
    first = ((i == 0) & (rI == 0) & (cI == 0)).astype(jnp.int32)
    v = v * (1 - first)
    vb = v.astype(jnp.bfloat16)

    r1 = lax.broadcasted_iota(jnp.int16, (128, 128), 0)
    c1 = lax.broadcasted_iota(jnp.int16, (128, 128), 1)
    U = jnp.where(r1 <= c1, jnp.bfloat16(1), jnp.bfloat16(0))
    lane_cum = jnp.dot(vb, U, preferred_element_type=jnp.float32)

    ones_col = jnp.full((128, 1), 1, jnp.bfloat16)
    rs = jnp.dot(vb, ones_col, preferred_element_type=jnp.float32)
    r2 = lax.broadcasted_iota(jnp.int16, (16, 16), 0)
    c2 = lax.broadcasted_iota(jnp.int16, (16, 16), 1)
    Ls = jnp.where(c2 < r2, jnp.bfloat16(1), jnp.bfloat16(0))
    excl = jnp.dot(Ls, rs.astype(jnp.bfloat16),
                   preferred_element_type=jnp.float32)

    s2d = (lane_cum + excl).astype(jnp.int32) + tot_ref[0]
    s_ref[0] = s2d
    tot_ref[0] = tot_ref[0] + jnp.sum(v)


def _cumsum(val):
    return pl.pallas_call(
        _k2_body,
        grid=(T_TOK // KBLK,),
        in_specs=[pl.BlockSpec((1, 16, 128), lambda i: (i, 0, 0))],
        out_specs=pl.BlockSpec((1, 16, 128), lambda i: (i, 0, 0)),
        out_shape=jax.ShapeDtypeStruct((T_TOK // KBLK, 16, 128), jnp.int32),
        scratch_shapes=[pltpu.SMEM((1,), jnp.int32)],
    )(val.reshape(T_TOK // KBLK, 16, 128))


def _k3_body(x2_hbm, p_hbm, s_hbm, out_hbm,
             idx_v, sv_v, rows0, rows1, gs0, gs1, ss0, ss1):
    wid = _wid()
    base = wid * TPW
    pltpu.sync_copy(x2_hbm.at[wid], idx_v)
    pltpu.sync_copy(s_hbm.at[pl.ds(base, TPW)], sv_v)

    rows = (rows0, rows1)
    gsem = (gs0, gs1)
    ssem = (ss0, ss1)

    def wait_gather(b):
        pltpu.make_async_copy(p_hbm.at[pl.ds(0, 128)], rows[b],
                              gsem[b]).wait()

    def wait_scatter(b, g):
        pltpu.make_async_copy(rows[b],
                              out_hbm.at[pl.ds(base + (g * 128), 128)],
                              ssem[b]).wait()

    def process(g, b):
        def tok16(j, carry):
            s_vec = sv_v[pl.ds(g * 128 + j * 16, 16)]
            for k in range(16):
                spl = jnp.full((16,), s_vec[k], jnp.int32)
                t = j * 16 + k
                for gg in range(8):
                    sl = (t, pl.ds(gg * 16, 16))
                    rows[b][sl] = rows[b][sl] + spl
            return carry

        lax.fori_loop(0, 8, tok16, jnp.int32(0))

    def body(k, carry):
        for b in range(2):
            g = k * 2 + b
            wait_gather(b)
            process(g, b)
            pltpu.async_copy(rows[b], out_hbm.at[pl.ds(base + (g * 128), 128)],
                             ssem[b])
            wait_scatter(b, g)
            pltpu.async_copy(p_hbm.at[idx_v.at[g + 2]], rows[b], gsem[b])
        return carry

    for b in range(2):
        pltpu.async_copy(p_hbm.at[idx_v.at[b]], rows[b], gsem[b])
    lax.fori_loop(0, (NGRP - 2) // 2, body, jnp.int32(0))
    for b in range(2):
        g = NGRP - 2 + b
        wait_gather(b)
        process(g, b)
        pltpu.async_copy(rows[b], out_hbm.at[pl.ds(base + (g * 128), 128)],
                         ssem[b])
        wait_scatter(b, g)


def kernel(x, table):
    B, Lx = x.shape
    p_arr, meta = _prep(table)
    vt = _valtab(meta, meta[:, 1].reshape(1, 1000)).reshape(1000000)
    x_flat = x.reshape(T_TOK)
    x_prev = jnp.concatenate([x_flat[:1], x_flat[:-1]])
    x2 = x_flat.reshape(NW, NGRP, 128)
    xp2 = x_prev.reshape(NW, NGRP, 128)

    mesh = plsc.VectorSubcoreMesh(core_axis_name="c", subcore_axis_name="s")

    k1 = pl.kernel(
        _k1_body,
        mesh=mesh,
        out_type=jax.ShapeDtypeStruct((T_TOK,), jnp.int32),
        scratch_types=[
            pltpu.VMEM((NGRP, 128), jnp.int32),
            pltpu.VMEM((NGRP, 128), jnp.int32),
            pltpu.VMEM((NGRP, 128), jnp.int32),
            pltpu.VMEM((TPW,), jnp.int32),
            pltpu.SemaphoreType.DMA,
        ],
    )
    val = k1(x2, xp2, vt)

    s_tok = _cumsum(val).reshape(T_TOK)

    k3 = pl.kernel(
        _k3_body,
        mesh=mesh,
        out_type=jax.ShapeDtypeStruct((T_TOK, 128), jnp.int32),
        scratch_types=[
            pltpu.VMEM((NGRP, 128), jnp.int32),
            pltpu.VMEM((TPW,), jnp.int32),
            pltpu.VMEM((128, 128), jnp.int32),
            pltpu.VMEM((128, 128), jnp.int32),
            pltpu.SemaphoreType.DMA,
            pltpu.SemaphoreType.DMA,
            pltpu.SemaphoreType.DMA,
            pltpu.SemaphoreType.DMA,
        ],
    )
    out = k3(x2, p_arr, s_tok)
    return out.reshape(B, Lx, 128)


# speedup vs baseline: 3.2892x; 1.0268x over previous
import jax
import jax.numpy as jnp
from jax import lax
from jax.experimental import pallas as pl
from jax.experimental.pallas import tpu as pltpu
from jax.experimental.pallas import tpu_sc as plsc

T_TOK = 204800
NW = 32
TPW = T_TOK // NW
NGRP = TPW // 128
KBLK = 4096


def _prep_body(tbl_ref, p_ref, meta_ref):
    tbl = tbl_ref[...]
    shifted = jnp.concatenate([tbl[:, :1], tbl[:, :127]], axis=1)
    ne = (tbl != shifted).astype(jnp.float32)
    r = lax.broadcasted_iota(jnp.int32, (128, 128), 0)
    c = lax.broadcasted_iota(jnp.int32, (128, 128), 1)
    M = (r <= c).astype(jnp.float32)
    P = jnp.dot(ne, M, preferred_element_type=jnp.float32)
    p_ref[...] = P.astype(jnp.int32)
    meta_ref[...] = jnp.zeros((1000, 8), jnp.int32)
    meta_ref[:, 0:1] = P[:, 127:128].astype(jnp.int32)
    meta_ref[:, 1:2] = lax.bitcast_convert_type(tbl[:, 0:1], jnp.int32)
    meta_ref[:, 2:3] = lax.bitcast_convert_type(tbl[:, 127:128], jnp.int32)


def _prep(table):
    return pl.pallas_call(
        _prep_body,
        out_shape=(
            jax.ShapeDtypeStruct((1000, 128), jnp.int32),
            jax.ShapeDtypeStruct((1000, 8), jnp.int32),
        ),
    )(table)


def _vt_body(meta_ref, frow_ref, vt_ref):
    c_col = meta_ref[:, 0:1]
    l_col = meta_ref[:, 2:3]
    f_row = frow_ref[...]
    vt_ref[...] = c_col + jnp.where(l_col != f_row, 1, 0)


def _valtab(meta, f_row):
    return pl.pallas_call(
        _vt_body,
        out_shape=jax.ShapeDtypeStruct((1000, 1000), jnp.int32),
    )(meta, f_row)


def _wid():
    return lax.axis_index("s") * 2 + lax.axis_index("c")


def _k1_body(x2_hbm, xp2_hbm, vt_hbm, val_hbm,
             idx_v, idxp_v, pidx_v, vv_v, sem):
    wid = _wid()
    base = wid * TPW
    pltpu.sync_copy(x2_hbm.at[wid], idx_v)
    pltpu.sync_copy(xp2_hbm.at[wid], idxp_v)

    def mkidx(g, carry):
        def sub(j, carry2):
            sl = (g, pl.ds(j * 16, 16))
            pidx_v[sl] = idxp_v[sl] * 1000 + idx_v[sl]
            return carry2
        return lax.fori_loop(0, 8, sub, carry)

    lax.fori_loop(0, NGRP, mkidx, jnp.int32(0))

    for g in range(NGRP):
        pltpu.async_copy(vt_hbm.at[pidx_v.at[g]], vv_v.at[pl.ds(g * 128, 128)],
                         sem)
    for g in range(NGRP):
        pltpu.make_async_copy(vt_hbm.at[pidx_v.at[g]],
                              vv_v.at[pl.ds(g * 128, 128)], sem).wait()
    pltpu.sync_copy(vv_v, val_hbm.at[pl.ds(base, TPW)])


def _k2_body(v_ref, s_ref, tot_ref):
    i = pl.program_id(0)

    @pl.when(i == 0)
    def _():
        tot_ref[0] = 0

    v = v_ref[0]
    rI = lax.broadcasted_iota(jnp.int32, (32, 128), 0)
    cI = lax.broadcasted_iota(jnp.int32, (32, 128), 1)
    first = ((i == 0) & (rI == 0) & (cI == 0)).astype(jnp.int32)
    v = v * (1 - first)
    vb = v.astype(jnp.bfloat16)

    r1 = lax.broadcasted_iota(jnp.int16, (128, 128), 0)
    c1 = lax.broadcasted_iota(jnp.int16, (128, 128), 1)
    U = jnp.where(r1 <= c1, jnp.bfloat16(1), jnp.bfloat16(0))
    lane_cum = jnp.dot(vb, U, preferred_element_type=jnp.float32)

    ones_col = jnp.full((128, 1), 1, jnp.bfloat16)
    rs = jnp.dot(vb, ones_col, preferred_element_type=jnp.float32)
    r2 = lax.broadcasted_iota(jnp.int16, (32, 32), 0)
    c2 = lax.broadcasted_iota(jnp.int16, (32, 32), 1)
    Ls = jnp.where(c2 < r2, jnp.bfloat16(1), jnp.bfloat16(0))
    excl = jnp.dot(Ls, rs.astype(jnp.bfloat16),
                   preferred_element_type=jnp.float32)

    s2d = (lane_cum + excl).astype(jnp.int32) + tot_ref[0]
    s_ref[0] = s2d
    tot_ref[0] = tot_ref[0] + jnp.sum(v)


def _cumsum(val):
    return pl.pallas_call(
        _k2_body,
        grid=(T_TOK // KBLK,),
        in_specs=[pl.BlockSpec((1, 32, 128), lambda i: (i, 0, 0))],
        out_specs=pl.BlockSpec((1, 32, 128), lambda i: (i, 0, 0)),
        out_shape=jax.ShapeDtypeStruct((T_TOK // KBLK, 32, 128), jnp.int32),
        scratch_shapes=[pltpu.SMEM((1,), jnp.int32)],
    )(val.reshape(T_TOK // KBLK, 32, 128))


def _k3_body(x2_hbm, p_hbm, s_hbm, out_hbm,
             idx_v, sv_v, rows0, rows1, gs0, gs1, ss0, ss1):
    wid = _wid()
    base = wid * TPW
    pltpu.sync_copy(x2_hbm.at[wid], idx_v)
    pltpu.sync_copy(s_hbm.at[pl.ds(base, TPW)], sv_v)

    rows = (rows0, rows1)
    gsem = (gs0, gs1)
    ssem = (ss0, ss1)

    def wait_gather(b):
        pltpu.make_async_copy(p_hbm.at[pl.ds(0, 128)], rows[b],
                              gsem[b]).wait()

    def wait_scatter(b, g):
        pltpu.make_async_copy(rows[b],
                              out_hbm.at[pl.ds(base + (g * 128), 128)],
                              ssem[b]).wait()

    def process(g, b):
        for j in range(8):
            s_vec = sv_v[pl.ds(g * 128 + j * 16, 16)]
            for k in range(16):
                spl = jnp.full((16,), s_vec[k], jnp.int32)
                t = j * 16 + k
                for gg in range(8):
                    sl = (t, pl.ds(gg * 16, 16))
                    rows[b][sl] = rows[b][sl] + spl

    def body(k, carry):
        for b in range(2):
            g = k * 2 + b
            wait_gather(b)
            process(g, b)
            pltpu.async_copy(rows[b], out_hbm.at[pl.ds(base + (g * 128), 128)],
                             ssem[b])
            wait_scatter(b, g)
            pltpu.async_copy(p_hbm.at[idx_v.at[g + 2]], rows[b], gsem[b])
        return carry

    for b in range(2):
        pltpu.async_copy(p_hbm.at[idx_v.at[b]], rows[b], gsem[b])
    lax.fori_loop(0, (NGRP - 2) // 2, body, jnp.int32(0))
    for b in range(2):
        g = NGRP - 2 + b
        wait_gather(b)
        process(g, b)
        pltpu.async_copy(rows[b], out_hbm.at[pl.ds(base + (g * 128), 128)],
                         ssem[b])
        wait_scatter(b, g)


def kernel(x, table):
    B, Lx = x.shape
    p_arr, meta = _prep(table)
    vt = _valtab(meta, meta[:, 1].reshape(1, 1000)).reshape(1000000)
    x_flat = x.reshape(T_TOK)
    x_prev = jnp.concatenate([x_flat[:1], x_flat[:-1]])
    x2 = x_flat.reshape(NW, NGRP, 128)
    xp2 = x_prev.reshape(NW, NGRP, 128)

    mesh = plsc.VectorSubcoreMesh(core_axis_name="c", subcore_axis_name="s")

    k1 = pl.kernel(
        _k1_body,
        mesh=mesh,
        out_type=jax.ShapeDtypeStruct((T_TOK,), jnp.int32),
        scratch_types=[
            pltpu.VMEM((NGRP, 128), jnp.int32),
            pltpu.VMEM((NGRP, 128), jnp.int32),
            pltpu.VMEM((NGRP, 128), jnp.int32),
            pltpu.VMEM((TPW,), jnp.int32),
            pltpu.SemaphoreType.DMA,
        ],
    )
    val = k1(x2, xp2, vt)

    s_tok = _cumsum(val).reshape(T_TOK)

    k3 = pl.kernel(
        _k3_body,
        mesh=mesh,
        out_type=jax.ShapeDtypeStruct((T_TOK, 128), jnp.int32),
        scratch_types=[
            pltpu.VMEM((NGRP, 128), jnp.int32),
            pltpu.VMEM((TPW,), jnp.int32),
            pltpu.VMEM((128, 128), jnp.int32),
            pltpu.VMEM((128, 128), jnp.int32),
            pltpu.SemaphoreType.DMA,
            pltpu.SemaphoreType.DMA,
            pltpu.SemaphoreType.DMA,
            pltpu.SemaphoreType.DMA,
        ],
    )
    out = k3(x2, p_arr, s_tok)
    return out.reshape(B, Lx, 128)


# speedup vs baseline: 3.6329x; 1.1045x over previous
import jax
import jax.numpy as jnp
from jax import lax
from jax.experimental import pallas as pl
from jax.experimental.pallas import tpu as pltpu
from jax.experimental.pallas import tpu_sc as plsc

T_TOK = 204800
NW = 32
TPW = T_TOK // NW
NGRP = TPW // 128
KBLK = 4096


def _prep_body(tbl_ref, p_ref, meta_ref):
    tbl = tbl_ref[...]
    shifted = jnp.concatenate([tbl[:, :1], tbl[:, :127]], axis=1)
    ne = (tbl != shifted).astype(jnp.float32)
    r = lax.broadcasted_iota(jnp.int32, (128, 128), 0)
    c = lax.broadcasted_iota(jnp.int32, (128, 128), 1)
    M = (r <= c).astype(jnp.float32)
    P = jnp.dot(ne, M, preferred_element_type=jnp.float32)
    p_ref[...] = P.astype(jnp.int32)
    meta_ref[...] = jnp.zeros((1000, 8), jnp.int32)
    meta_ref[:, 0:1] = P[:, 127:128].astype(jnp.int32)
    meta_ref[:, 1:2] = lax.bitcast_convert_type(tbl[:, 0:1], jnp.int32)
    meta_ref[:, 2:3] = lax.bitcast_convert_type(tbl[:, 127:128], jnp.int32)


def _prep(table):
    return pl.pallas_call(
        _prep_body,
        out_shape=(
            jax.ShapeDtypeStruct((1000, 128), jnp.int32),
            jax.ShapeDtypeStruct((1000, 8), jnp.int32),
        ),
    )(table)


def _vt_body(meta_ref, frow_ref, vt_ref):
    c_col = meta_ref[:, 0:1]
    l_col = meta_ref[:, 2:3]
    f_row = frow_ref[...]
    vt_ref[...] = c_col + jnp.where(l_col != f_row, 1, 0)


def _valtab(meta, f_row):
    return pl.pallas_call(
        _vt_body,
        out_shape=jax.ShapeDtypeStruct((1000, 1000), jnp.int32),
    )(meta, f_row)


def _wid():
    return lax.axis_index("s") * 2 + lax.axis_index("c")


def _k1_body(x2_hbm, xp2_hbm, vt_hbm, val_hbm,
             idx_v, idxp_v, pidx_v, vv_v, sem):
    wid = _wid()
    base = wid * TPW
    pltpu.sync_copy(x2_hbm.at[wid], idx_v)
    pltpu.sync_copy(xp2_hbm.at[wid], idxp_v)

    def mkidx(g, carry):
        def sub(j, carry2):
            sl = (g, pl.ds(j * 16, 16))
            pidx_v[sl] = idxp_v[sl] * 1000 + idx_v[sl]
            return carry2
        return lax.fori_loop(0, 8, sub, carry)

    lax.fori_loop(0, NGRP, mkidx, jnp.int32(0))

    for g in range(NGRP):
        pltpu.async_copy(vt_hbm.at[pidx_v.at[g]], vv_v.at[pl.ds(g * 128, 128)],
                         sem)
    for g in range(NGRP):
        pltpu.make_async_copy(vt_hbm.at[pidx_v.at[g]],
                              vv_v.at[pl.ds(g * 128, 128)], sem).wait()
    pltpu.sync_copy(vv_v, val_hbm.at[pl.ds(base, TPW)])


def _k2_body(v_ref, s_ref, tot_ref):
    i = pl.program_id(0)

    @pl.when(i == 0)
    def _():
        tot_ref[0] = 0

    v = v_ref[0]
    rI = lax.broadcasted_iota(jnp.int32, (32, 128), 0)
    cI = lax.broadcasted_iota(jnp.int32, (32, 128), 1)
    first = ((i == 0) & (rI == 0) & (cI == 0)).astype(jnp.int32)
    v = v * (1 - first)
    vb = v.astype(jnp.bfloat16)

    r1 = lax.broadcasted_iota(jnp.int16, (128, 128), 0)
    c1 = lax.broadcasted_iota(jnp.int16, (128, 128), 1)
    U = jnp.where(r1 <= c1, jnp.bfloat16(1), jnp.bfloat16(0))
    lane_cum = jnp.dot(vb, U, preferred_element_type=jnp.float32)

    ones_col = jnp.full((128, 1), 1, jnp.bfloat16)
    rs = jnp.dot(vb, ones_col, preferred_element_type=jnp.float32)
    r2 = lax.broadcasted_iota(jnp.int16, (32, 32), 0)
    c2 = lax.broadcasted_iota(jnp.int16, (32, 32), 1)
    Ls = jnp.where(c2 < r2, jnp.bfloat16(1), jnp.bfloat16(0))
    excl = jnp.dot(Ls, rs.astype(jnp.bfloat16),
                   preferred_element_type=jnp.float32)

    s2d = (lane_cum + excl).astype(jnp.int32) + tot_ref[0]
    s_ref[0] = s2d
    tot_ref[0] = tot_ref[0] + jnp.sum(v)


def _cumsum(val):
    return pl.pallas_call(
        _k2_body,
        grid=(T_TOK // KBLK,),
        in_specs=[pl.BlockSpec((1, 32, 128), lambda i: (i, 0, 0))],
        out_specs=pl.BlockSpec((1, 32, 128), lambda i: (i, 0, 0)),
        out_shape=jax.ShapeDtypeStruct((T_TOK // KBLK, 32, 128), jnp.int32),
        scratch_shapes=[pltpu.SMEM((1,), jnp.int32)],
    )(val.reshape(T_TOK // KBLK, 32, 128))


def _k3_body(x2_hbm, p_hbm, s_hbm, out_hbm,
             idx_v, sv_v, rows0, rows1, gs0, gs1, ss0, ss1):
    wid = _wid()
    base = wid * TPW
    pltpu.sync_copy(x2_hbm.at[wid], idx_v)
    pltpu.sync_copy(s_hbm.at[pl.ds(base, TPW)], sv_v)

    rows = (rows0, rows1)
    gsem = (gs0, gs1)
    ssem = (ss0, ss1)

    def wait_gather(b):
        pltpu.make_async_copy(p_hbm.at[pl.ds(0, 128)], rows[b],
                              gsem[b]).wait()

    def wait_scatter(b, g):
        pltpu.make_async_copy(rows[b],
                              out_hbm.at[pl.ds(base + (g * 128), 128)],
                              ssem[b]).wait()

    def process(g, b):
        def tok16(j, carry):
            s_vec = sv_v[pl.ds(g * 128 + j * 16, 16)]
            for k in range(16):
                spl = jnp.full((16,), s_vec[k], jnp.int32)
                t = j * 16 + k
                for gg in range(8):
                    sl = (t, pl.ds(gg * 16, 16))
                    rows[b][sl] = rows[b][sl] + spl
            return carry

        lax.fori_loop(0, 8, tok16, jnp.int32(0))

    def body(k, carry):
        for b in range(2):
            g = k * 2 + b
            wait_gather(b)
            process(g, b)
            pltpu.async_copy(rows[b], out_hbm.at[pl.ds(base + (g * 128), 128)],
                             ssem[b])
            wait_scatter(b, g)
            pltpu.async_copy(p_hbm.at[idx_v.at[g + 2]], rows[b], gsem[b])
        return carry

    for b in range(2):
        pltpu.async_copy(p_hbm.at[idx_v.at[b]], rows[b], gsem[b])
    lax.fori_loop(0, (NGRP - 2) // 2, body, jnp.int32(0))
    for b in range(2):
        g = NGRP - 2 + b
        wait_gather(b)
        process(g, b)
        pltpu.async_copy(rows[b], out_hbm.at[pl.ds(base + (g * 128), 128)],
                         ssem[b])
        wait_scatter(b, g)


def kernel(x, table):
    B, Lx = x.shape
    p_arr, meta = _prep(table)
    vt = _valtab(meta, meta[:, 1].reshape(1, 1000)).reshape(1000000)
    x_flat = x.reshape(T_TOK)
    x_prev = jnp.concatenate([x_flat[:1], x_flat[:-1]])
    x2 = x_flat.reshape(NW, NGRP, 128)
    xp2 = x_prev.reshape(NW, NGRP, 128)

    mesh = plsc.VectorSubcoreMesh(core_axis_name="c", subcore_axis_name="s")

    k1 = pl.kernel(
        _k1_body,
        mesh=mesh,
        out_type=jax.ShapeDtypeStruct((T_TOK,), jnp.int32),
        scratch_types=[
            pltpu.VMEM((NGRP, 128), jnp.int32),
            pltpu.VMEM((NGRP, 128), jnp.int32),
            pltpu.VMEM((NGRP, 128), jnp.int32),
            pltpu.VMEM((TPW,), jnp.int32),
            pltpu.SemaphoreType.DMA,
        ],
    )
    val = k1(x2, xp2, vt)

    s_tok = _cumsum(val).reshape(T_TOK)

    k3 = pl.kernel(
        _k3_body,
        mesh=mesh,
        out_type=jax.ShapeDtypeStruct((T_TOK, 128), jnp.int32),
        scratch_types=[
            pltpu.VMEM((NGRP, 128), jnp.int32),
            pltpu.VMEM((TPW,), jnp.int32),
            pltpu.VMEM((128, 128), jnp.int32),
            pltpu.VMEM((128, 128), jnp.int32),
            pltpu.SemaphoreType.DMA,
            pltpu.SemaphoreType.DMA,
            pltpu.SemaphoreType.DMA,
            pltpu.SemaphoreType.DMA,
        ],
    )
    out = k3(x2, p_arr, s_tok)
    return out.reshape(B, Lx, 128)
